# baseline (speedup 1.0000x reference)
import functools

import jax
import jax.numpy as jnp
from jax import lax
from jax.experimental import pallas as pl
from jax.experimental.pallas import tpu as pltpu
from jax.experimental.pallas import tpu_sc as plsc

_B = 4
_N = 2048
_G = 2
_RPT = 16 * _G
_SCROWS = _RPT * 8
_NTC = _N - _SCROWS


def _sqrt16(x):
    x = jnp.maximum(x, jnp.float32(1e-24))
    i = lax.bitcast_convert_type(x, jnp.int32)
    i = jnp.int32(0x5F3759DF) - (i >> 1)
    y = lax.bitcast_convert_type(i, jnp.float32)
    h = jnp.float32(0.5) * x
    for _ in range(3):
        y = y * (jnp.float32(1.5) - h * y * y)
    return x * y


def _round_bf16(x):
    i = lax.bitcast_convert_type(x, jnp.int32)
    i = i + jnp.int32(0x7FFF) + ((i >> 16) & jnp.int32(1))
    i = i & jnp.int32(-65536)
    return lax.bitcast_convert_type(i, jnp.float32)


def _take16(v, idx):
    return lax.gather(
        v, idx[:, None],
        lax.GatherDimensionNumbers(offset_dims=(), collapsed_slice_dims=(0,),
                                   start_index_map=(0,)),
        (1,), mode=lax.GatherScatterMode.PROMISE_IN_BOUNDS)


def _tree_min(vs):
    while len(vs) > 1:
        vs = [jnp.minimum(vs[i], vs[i + 1]) for i in range(0, len(vs) - 1, 2)] \
            + ([vs[-1]] if len(vs) % 2 else [])
    return vs[0]


_INF16 = lambda: jnp.full((16,), jnp.float32(jnp.inf))


def _sc_body(a1_hbm, a2_hbm, rows_out, cols_out,
             xv, yv, yr, y2b, xr, x2b, colp, tmp8, cmb, shared):
    cid = lax.axis_index("c")
    sid = lax.axis_index("s")
    b = cid * 2 + sid // 8
    slot = sid % 8
    r0 = slot * _RPT

    pltpu.sync_copy(a1_hbm.at[b], xv)
    pltpu.sync_copy(a2_hbm.at[b], yv)

    def prep_y(k, carry):
        s = k * 16
        v0 = yv[0, pl.ds(s, 16)]
        v1 = yv[1, pl.ds(s, 16)]
        v2 = yv[2, pl.ds(s, 16)]
        q0 = _round_bf16(v0)
        q1 = _round_bf16(v1)
        q2 = _round_bf16(v2)
        yr[0, pl.ds(s, 16)] = q0 + q0
        yr[1, pl.ds(s, 16)] = q1 + q1
        yr[2, pl.ds(s, 16)] = q2 + q2
        y2b[pl.ds(s, 16)] = v0 * v0 + v1 * v1 + v2 * v2
        colp[pl.ds(s, 16)] = _INF16()
        return carry

    lax.fori_loop(0, _N // 16, prep_y, 0)

    def prep_x(k, carry):
        s = k * 16
        v0 = xv[0, pl.ds(r0 + s, 16)]
        v1 = xv[1, pl.ds(r0 + s, 16)]
        v2 = xv[2, pl.ds(r0 + s, 16)]
        xr[0, pl.ds(s, 16)] = _round_bf16(v0)
        xr[1, pl.ds(s, 16)] = _round_bf16(v1)
        xr[2, pl.ds(s, 16)] = _round_bf16(v2)
        x2b[pl.ds(s, 16)] = v0 * v0 + v1 * v1 + v2 * v2
        return carry

    lax.fori_loop(0, _RPT // 16, prep_x, 0)

    lanes = lax.iota(jnp.int32, 16)
    xors = [lanes ^ jnp.int32(k) for k in (8, 4, 2, 1)]

    xs = []
    x2s = []
    for gi in range(_G):
        off = gi * 16
        xs.append((xr[0, pl.ds(off, 16)],
                   xr[1, pl.ds(off, 16)],
                   xr[2, pl.ds(off, 16)]))
        x2s.append(x2b[pl.ds(off, 16)])

    def mbody(mc, accs):
        s = mc * 16
        o0 = yr[0, pl.ds(s, 16)]
        o1 = yr[1, pl.ds(s, 16)]
        o2 = yr[2, pl.ds(s, 16)]
        oy2 = y2b[pl.ds(s, 16)]
        accs = list(accs)
        colv = _INF16()
        for j in range(16):
            jc = jnp.full((16,), j, jnp.int32)
            b0 = _take16(o0, jc)
            b1 = _take16(o1, jc)
            b2 = _take16(o2, jc)
            y2v = _take16(oy2, jc)
            parts = []
            for gi in range(_G):
                q0, q1, q2 = xs[gi]
                t = q0 * b0 + q1 * b1 + q2 * b2
                dd = (x2s[gi] + y2v) - t
                accs[gi] = jnp.minimum(accs[gi], dd)
                parts.append(dd)
            p = _tree_min(parts)
            for xk in xors:
                p = jnp.minimum(p, _take16(p, xk))
            colv = jnp.where(lanes == jnp.int32(j), p, colv)
        colp[pl.ds(s, 16)] = jnp.minimum(colp[pl.ds(s, 16)], colv)
        return tuple(accs)

    accs = lax.fori_loop(0, _N // 16, mbody,
                         tuple(_INF16() for _ in range(_G)))
    rowsum = jnp.zeros((16,), jnp.float32)
    for gi in range(_G):
        rowsum = rowsum + _sqrt16(accs[gi])

    pltpu.sync_copy(colp, shared.at[sid])
    plsc.subcore_barrier()
    sbase = (sid // 8) * 8
    m0 = slot * 256
    pltpu.sync_copy(shared.at[pl.ds(sbase, 8), pl.ds(m0, 256)], tmp8)

    for k in range(256 // 16):
        vs = [tmp8[p, pl.ds(k * 16, 16)] for p in range(8)]
        cmb[0, pl.ds(k * 16, 16)] = _tree_min(vs)
    cmb[1, pl.ds(0, 16)] = rowsum

    pltpu.sync_copy(cmb, cols_out.at[cid * 16 + sid])
    pltpu.sync_copy(cmb.at[1, pl.ds(0, 16)], rows_out.at[cid * 16 + sid])


def _sc_chamfer(a1t, a2t):
    mesh = plsc.VectorSubcoreMesh(core_axis_name="c", subcore_axis_name="s")
    run = pl.kernel(
        _sc_body,
        out_type=(jax.ShapeDtypeStruct((32, 16), jnp.float32),
                  jax.ShapeDtypeStruct((32, 2, 256), jnp.float32)),
        mesh=mesh,
        scratch_types=[
            pltpu.VMEM((3, _N), jnp.float32),
            pltpu.VMEM((3, _N), jnp.float32),
            pltpu.VMEM((3, _N), jnp.float32),
            pltpu.VMEM((_N,), jnp.float32),
            pltpu.VMEM((3, _RPT), jnp.float32),
            pltpu.VMEM((_RPT,), jnp.float32),
            pltpu.VMEM((_N,), jnp.float32),
            pltpu.VMEM((8, 256), jnp.float32),
            pltpu.VMEM((2, 256), jnp.float32),
            pltpu.VMEM_SHARED((16, _N), jnp.float32),
        ],
    )
    return run(a1t, a2t)


def _tc_round(x):
    return x.astype(jnp.bfloat16).astype(jnp.float32)


def _tc_main_body(a1_ref, a2t_ref, rowout_ref, colout_ref, ybuf):
    for mb in range(16):
        sl = pl.ds(mb * 128, 128)
        v0 = a2t_ref[0:1, sl]
        v1 = a2t_ref[1:2, sl]
        v2 = a2t_ref[2:3, sl]
        q0 = _tc_round(v0)
        q1 = _tc_round(v1)
        q2 = _tc_round(v2)
        ybuf[0:1, sl] = q0 + q0
        ybuf[1:2, sl] = q1 + q1
        ybuf[2:3, sl] = q2 + q2
        ybuf[3:4, sl] = v0 * v0 + v1 * v1 + v2 * v2

    inf8 = jnp.full((8, 128), jnp.float32(jnp.inf))

    def nblock(nb, carry):
        colaccs, rsum = carry
        n0 = _SCROWS + nb * 8
        x0 = a1_ref[pl.ds(n0, 8), 0:1]
        x1 = a1_ref[pl.ds(n0, 8), 1:2]
        x2c = a1_ref[pl.ds(n0, 8), 2:3]
        r0 = _tc_round(x0)
        r1 = _tc_round(x1)
        r2 = _tc_round(x2c)
        x2v = x0 * x0 + x1 * x1 + x2c * x2c
        rowacc = inf8
        new_cols = []
        for mb in range(16):
            sl = pl.ds(mb * 128, 128)
            t = r0 * ybuf[0:1, sl] + r1 * ybuf[1:2, sl] + r2 * ybuf[2:3, sl]
            dd = (x2v + ybuf[3:4, sl]) - t
            dd = jnp.maximum(dd, jnp.float32(0.0))
            rowacc = jnp.minimum(rowacc, dd)
            new_cols.append(jnp.minimum(colaccs[mb], dd))
        rmin = jnp.min(rowacc, axis=1, keepdims=True)
        rsum = rsum + jnp.sqrt(rmin)
        return tuple(new_cols), rsum

    colaccs, rsum = lax.fori_loop(
        0, _NTC // 8, nblock,
        (tuple(inf8 for _ in range(16)), jnp.zeros((8, 1), jnp.float32)))

    rowout_ref[...] = jnp.broadcast_to(rsum, (8, 128))
    for mb in range(16):
        colout_ref[mb] = colaccs[mb]


def _tc_main(a1, a2t):
    return pl.pallas_call(
        _tc_main_body,
        grid=(_B,),
        in_specs=[
            pl.BlockSpec((None, _N, 3), lambda b: (b, 0, 0)),
            pl.BlockSpec((None, 3, _N), lambda b: (b, 0, 0)),
        ],
        out_specs=[
            pl.BlockSpec((None, 8, 128), lambda b: (b, 0, 0)),
            pl.BlockSpec((None, 16, 8, 128), lambda b: (b, 0, 0, 0)),
        ],
        out_shape=[
            jax.ShapeDtypeStruct((_B, 8, 128), jnp.float32),
            jax.ShapeDtypeStruct((_B, 16, 8, 128), jnp.float32),
        ],
        scratch_shapes=[pltpu.VMEM((4, _N), jnp.float32)],
    )(a1, a2t)


def _tc_combine_body(scrow_ref, sccol_ref, tcrow_ref, tccol_ref, out_ref):
    total = jnp.sum(scrow_ref[...])
    total = total + jnp.sum(tcrow_ref[:, :, 0:1])
    for b in range(_B):
        for mb in range(16):
            tc8 = tccol_ref[b, mb]
            tcmin = jnp.min(tc8, axis=0, keepdims=True)
            scmin = sccol_ref[b, mb:mb + 1, :]
            comb = jnp.maximum(jnp.minimum(tcmin, scmin), jnp.float32(0.0))
            total = total + jnp.sum(jnp.sqrt(comb))
    out_ref[0, 0] = total / jnp.float32(2 * _B * _N)


def _tc_combine(scrow, sccol, tcrow, tccol):
    return pl.pallas_call(
        _tc_combine_body,
        out_shape=jax.ShapeDtypeStruct((1, 1), jnp.float32),
        out_specs=pl.BlockSpec(memory_space=pltpu.SMEM),
    )(scrow, sccol, tcrow, tccol)


def kernel(array1, array2):
    a1t = jnp.transpose(array1, (0, 2, 1))
    a2t = jnp.transpose(array2, (0, 2, 1))
    sc_rows, sc_cols = _sc_chamfer(a1t, a2t)
    tc_rows, tc_cols = _tc_main(array1, a2t)
    colpart = sc_cols[:, 0, :].reshape(2, 2, 8, 256)
    colpart = colpart.reshape(2, 2, _N).reshape(_B, 16, 128)
    out = _tc_combine(sc_rows, colpart, tc_rows, tc_cols)
    return out.reshape(())


# speedup vs baseline: 1.3220x; 1.3220x over previous
import functools

import jax
import jax.numpy as jnp
from jax import lax
from jax.experimental import pallas as pl
from jax.experimental.pallas import tpu as pltpu
from jax.experimental.pallas import tpu_sc as plsc

_B = 4
_N = 2048
_G = 2
_RPT = 16 * _G
_SCROWS = _RPT * 8
_NTC = _N - _SCROWS


def _sqrt16(x):
    x = jnp.maximum(x, jnp.float32(1e-24))
    i = lax.bitcast_convert_type(x, jnp.int32)
    i = jnp.int32(0x5F3759DF) - (i >> 1)
    y = lax.bitcast_convert_type(i, jnp.float32)
    h = jnp.float32(0.5) * x
    for _ in range(3):
        y = y * (jnp.float32(1.5) - h * y * y)
    return x * y


def _round_bf16(x):
    i = lax.bitcast_convert_type(x, jnp.int32)
    i = i + jnp.int32(0x7FFF) + ((i >> 16) & jnp.int32(1))
    i = i & jnp.int32(-65536)
    return lax.bitcast_convert_type(i, jnp.float32)


def _take16(v, idx):
    return lax.gather(
        v, idx[:, None],
        lax.GatherDimensionNumbers(offset_dims=(), collapsed_slice_dims=(0,),
                                   start_index_map=(0,)),
        (1,), mode=lax.GatherScatterMode.PROMISE_IN_BOUNDS)


def _tree_min(vs):
    while len(vs) > 1:
        vs = [jnp.minimum(vs[i], vs[i + 1]) for i in range(0, len(vs) - 1, 2)] \
            + ([vs[-1]] if len(vs) % 2 else [])
    return vs[0]


_INF16 = lambda: jnp.full((16,), jnp.float32(jnp.inf))


def _sc_body(a1_hbm, a2_hbm, rows_out, cols_out,
             xv, yv, yr, y2b, xr, x2b, colp, tmp8, cmb, shared):
    cid = lax.axis_index("c")
    sid = lax.axis_index("s")
    b = cid * 2 + sid // 8
    slot = sid % 8
    r0 = slot * _RPT

    pltpu.sync_copy(a1_hbm.at[b], xv)
    pltpu.sync_copy(a2_hbm.at[b], yv)

    def prep_y(k, carry):
        s = k * 16
        v0 = yv[0, pl.ds(s, 16)]
        v1 = yv[1, pl.ds(s, 16)]
        v2 = yv[2, pl.ds(s, 16)]
        q0 = _round_bf16(v0)
        q1 = _round_bf16(v1)
        q2 = _round_bf16(v2)
        yr[0, pl.ds(s, 16)] = q0 + q0
        yr[1, pl.ds(s, 16)] = q1 + q1
        yr[2, pl.ds(s, 16)] = q2 + q2
        y2b[pl.ds(s, 16)] = v0 * v0 + v1 * v1 + v2 * v2
        colp[pl.ds(s, 16)] = _INF16()
        return carry

    lax.fori_loop(0, _N // 16, prep_y, 0)

    def prep_x(k, carry):
        s = k * 16
        v0 = xv[0, pl.ds(r0 + s, 16)]
        v1 = xv[1, pl.ds(r0 + s, 16)]
        v2 = xv[2, pl.ds(r0 + s, 16)]
        xr[0, pl.ds(s, 16)] = _round_bf16(v0)
        xr[1, pl.ds(s, 16)] = _round_bf16(v1)
        xr[2, pl.ds(s, 16)] = _round_bf16(v2)
        x2b[pl.ds(s, 16)] = v0 * v0 + v1 * v1 + v2 * v2
        return carry

    lax.fori_loop(0, _RPT // 16, prep_x, 0)

    lanes = lax.iota(jnp.int32, 16)
    xors = [lanes ^ jnp.int32(k) for k in (8, 4, 2, 1)]

    xs = []
    x2s = []
    for gi in range(_G):
        off = gi * 16
        xs.append((xr[0, pl.ds(off, 16)],
                   xr[1, pl.ds(off, 16)],
                   xr[2, pl.ds(off, 16)]))
        x2s.append(x2b[pl.ds(off, 16)])

    def mbody(mc, accs):
        s = mc * 16
        o0 = yr[0, pl.ds(s, 16)]
        o1 = yr[1, pl.ds(s, 16)]
        o2 = yr[2, pl.ds(s, 16)]
        oy2 = y2b[pl.ds(s, 16)]
        accs = list(accs)
        colv = _INF16()
        for j in range(16):
            jc = jnp.full((16,), j, jnp.int32)
            b0 = _take16(o0, jc)
            b1 = _take16(o1, jc)
            b2 = _take16(o2, jc)
            y2v = _take16(oy2, jc)
            parts = []
            for gi in range(_G):
                q0, q1, q2 = xs[gi]
                t = q0 * b0 + q1 * b1 + q2 * b2
                dd = (x2s[gi] + y2v) - t
                accs[gi] = jnp.minimum(accs[gi], dd)
                parts.append(dd)
            p = _tree_min(parts)
            for xk in xors:
                p = jnp.minimum(p, _take16(p, xk))
            colv = jnp.where(lanes == jnp.int32(j), p, colv)
        colp[pl.ds(s, 16)] = jnp.minimum(colp[pl.ds(s, 16)], colv)
        return tuple(accs)

    accs = lax.fori_loop(0, _N // 16, mbody,
                         tuple(_INF16() for _ in range(_G)))
    rowsum = jnp.zeros((16,), jnp.float32)
    for gi in range(_G):
        rowsum = rowsum + _sqrt16(accs[gi])

    pltpu.sync_copy(colp, shared.at[sid])
    plsc.subcore_barrier()
    sbase = (sid // 8) * 8
    m0 = slot * 256
    pltpu.sync_copy(shared.at[pl.ds(sbase, 8), pl.ds(m0, 256)], tmp8)

    for k in range(256 // 16):
        vs = [tmp8[p, pl.ds(k * 16, 16)] for p in range(8)]
        cmb[0, pl.ds(k * 16, 16)] = _tree_min(vs)
    cmb[1, pl.ds(0, 16)] = rowsum

    pltpu.sync_copy(cmb, cols_out.at[cid * 16 + sid])
    pltpu.sync_copy(cmb.at[1, pl.ds(0, 16)], rows_out.at[cid * 16 + sid])


def _sc_chamfer(a1t, a2t):
    mesh = plsc.VectorSubcoreMesh(core_axis_name="c", subcore_axis_name="s")
    run = pl.kernel(
        _sc_body,
        out_type=(jax.ShapeDtypeStruct((32, 16), jnp.float32),
                  jax.ShapeDtypeStruct((32, 2, 256), jnp.float32)),
        mesh=mesh,
        scratch_types=[
            pltpu.VMEM((3, _N), jnp.float32),
            pltpu.VMEM((3, _N), jnp.float32),
            pltpu.VMEM((3, _N), jnp.float32),
            pltpu.VMEM((_N,), jnp.float32),
            pltpu.VMEM((3, _RPT), jnp.float32),
            pltpu.VMEM((_RPT,), jnp.float32),
            pltpu.VMEM((_N,), jnp.float32),
            pltpu.VMEM((8, 256), jnp.float32),
            pltpu.VMEM((2, 256), jnp.float32),
            pltpu.VMEM_SHARED((16, _N), jnp.float32),
        ],
    )
    return run(a1t, a2t)


def _tc_round(x):
    return x.astype(jnp.bfloat16).astype(jnp.float32)


_W = 256
_XB = _NTC // _W


def _tc_main_body(xp_ref, ybc_ref, rowout_ref, ypart_ref, bufa, bufb):
    inf8 = jnp.full((8, 128), jnp.float32(jnp.inf))
    nyb = _N // 8

    def initblk(yb, c):
        bufa[pl.ds(yb * 8, 8), :] = inf8
        return c

    lax.fori_loop(0, nyb, initblk, 0)

    bufs = [bufa, bufb]
    xsum = jnp.zeros((1, 128), jnp.float32)
    for xb in range(_XB):
        src = bufs[xb % 2]
        dst = bufs[1 - xb % 2]
        xq = [jnp.broadcast_to(xp_ref[c:c + 1, pl.ds(xb * _W, _W)], (8, _W))
              for c in range(4)]
        xh = [[q[:, h * 128:(h + 1) * 128] for q in xq] for h in range(2)]

        def ystep(yb, xaccs, xh=xh, src=src, dst=dst):
            rsl = pl.ds(yb * 8, 8)
            yb0 = ybc_ref[0, rsl, :]
            yb1 = ybc_ref[1, rsl, :]
            yb2 = ybc_ref[2, rsl, :]
            y2v = ybc_ref[3, rsl, :]
            dds = []
            new_accs = []
            for h in range(2):
                q0, q1, q2, x2v = xh[h]
                t = q0 * yb0 + q1 * yb1 + q2 * yb2
                dd = (x2v + y2v) - t
                dds.append(dd)
                new_accs.append(jnp.minimum(xaccs[h], dd))
            m = jnp.minimum(dds[0], dds[1])
            dst[rsl, :] = jnp.minimum(src[rsl, :], m)
            return tuple(new_accs)

        xaccs = lax.fori_loop(0, nyb, ystep, (inf8, inf8))
        for h in range(2):
            red = jnp.min(xaccs[h], axis=0, keepdims=True)
            red = jnp.maximum(red, jnp.float32(0.0))
            xsum = xsum + jnp.sqrt(red)

    rowout_ref[...] = jnp.broadcast_to(xsum, (8, 128))

    fin = bufs[1 - (_XB - 1) % 2]
    def yfinal(yb, c):
        v = fin[pl.ds(yb * 8, 8), :]
        ypart_ref[pl.ds(yb * 8, 8), 0:1] = jnp.min(v, axis=1, keepdims=True)
        return c

    lax.fori_loop(0, nyb, yfinal, 0)


def _tc_main(a1, a2t):
    a1t = jnp.transpose(a1, (0, 2, 1))[:, :, _SCROWS:]
    xr = _tc_round(a1t)
    xp = jnp.concatenate(
        [xr + xr, jnp.sum(a1t * a1t, axis=1, keepdims=True)], axis=1)
    yvals = jnp.concatenate(
        [_tc_round(a2t), jnp.sum(a2t * a2t, axis=1, keepdims=True)], axis=1)
    ybc = jnp.broadcast_to(yvals[:, :, :, None], (_B, 4, _N, 128))
    return pl.pallas_call(
        _tc_main_body,
        grid=(_B,),
        in_specs=[
            pl.BlockSpec((None, 4, _NTC), lambda b: (b, 0, 0)),
            pl.BlockSpec((None, 4, _N, 128), lambda b: (b, 0, 0, 0)),
        ],
        out_specs=[
            pl.BlockSpec((None, 8, 128), lambda b: (b, 0, 0)),
            pl.BlockSpec((None, _N, 1), lambda b: (b, 0, 0)),
        ],
        out_shape=[
            jax.ShapeDtypeStruct((_B, 8, 128), jnp.float32),
            jax.ShapeDtypeStruct((_B, _N, 1), jnp.float32),
        ],
        scratch_shapes=[
            pltpu.VMEM((_N, 128), jnp.float32),
            pltpu.VMEM((_N, 128), jnp.float32),
        ],
    )(xp, ybc)


def _tc_combine_body(scrow_ref, sccol_ref, tcrow_ref, tccol_ref, out_ref):
    total = jnp.sum(scrow_ref[...])
    total = total + jnp.sum(tcrow_ref[:, 0:1, :])
    for b in range(_B):
        for mb in range(16):
            tcmin = tccol_ref[b, mb:mb + 1, :]
            scmin = sccol_ref[b, mb:mb + 1, :]
            comb = jnp.maximum(jnp.minimum(tcmin, scmin), jnp.float32(0.0))
            total = total + jnp.sum(jnp.sqrt(comb))
    out_ref[0, 0] = total / jnp.float32(2 * _B * _N)


def _tc_combine(scrow, sccol, tcrow, tccol):
    return pl.pallas_call(
        _tc_combine_body,
        out_shape=jax.ShapeDtypeStruct((1, 1), jnp.float32),
        out_specs=pl.BlockSpec(memory_space=pltpu.SMEM),
    )(scrow, sccol, tcrow, tccol)


def kernel(array1, array2):
    a1t = jnp.transpose(array1, (0, 2, 1))
    a2t = jnp.transpose(array2, (0, 2, 1))
    sc_rows, sc_cols = _sc_chamfer(a1t, a2t)
    tc_rows, tc_cols = _tc_main(array1, a2t)
    colpart = sc_cols[:, 0, :].reshape(2, 2, 8, 256)
    colpart = colpart.reshape(2, 2, _N).reshape(_B, 16, 128)
    tccol = tc_cols.reshape(_B, _N).reshape(_B, 16, 128)
    out = _tc_combine(sc_rows, colpart, tc_rows, tccol)
    return out.reshape(())


# speedup vs baseline: 2.3198x; 1.7548x over previous
import jax
import jax.numpy as jnp
from jax import lax
from jax.experimental import pallas as pl
from jax.experimental.pallas import tpu as pltpu
from jax.experimental.pallas import tpu_sc as plsc

_B = 4
_N = 2048
_RPT = 256
_G = 8
_HALVES = _RPT // (16 * _G)


def _sqrt16(x):
    x = jnp.maximum(x, jnp.float32(1e-24))
    i = lax.bitcast_convert_type(x, jnp.int32)
    i = jnp.int32(0x5F3759DF) - (i >> 1)
    y = lax.bitcast_convert_type(i, jnp.float32)
    h = jnp.float32(0.5) * x
    for _ in range(3):
        y = y * (jnp.float32(1.5) - h * y * y)
    return x * y


def _round_bf16(x):
    i = lax.bitcast_convert_type(x, jnp.int32)
    i = i + jnp.int32(0x7FFF) + ((i >> 16) & jnp.int32(1))
    i = i & jnp.int32(-65536)
    return lax.bitcast_convert_type(i, jnp.float32)


def _take16(v, idx):
    return lax.gather(
        v, idx[:, None],
        lax.GatherDimensionNumbers(offset_dims=(), collapsed_slice_dims=(0,),
                                   start_index_map=(0,)),
        (1,), mode=lax.GatherScatterMode.PROMISE_IN_BOUNDS)


def _tree_min(vs):
    while len(vs) > 1:
        vs = [jnp.minimum(vs[i], vs[i + 1]) for i in range(0, len(vs) - 1, 2)] \
            + ([vs[-1]] if len(vs) % 2 else [])
    return vs[0]


_INF16 = lambda: jnp.full((16,), jnp.float32(jnp.inf))


def _sc_body(a1_hbm, a2_hbm, out_hbm,
             xv, yv, yr, y2b, xr, x2b, colp, tmp8, accv, shared):
    cid = lax.axis_index("c")
    sid = lax.axis_index("s")
    b = cid * 2 + sid // 8
    slot = sid % 8
    r0 = slot * _RPT

    pltpu.sync_copy(a1_hbm.at[b], xv)
    pltpu.sync_copy(a2_hbm.at[b], yv)

    def prep_y(k, carry):
        s = k * 16
        v0 = yv[0, pl.ds(s, 16)]
        v1 = yv[1, pl.ds(s, 16)]
        v2 = yv[2, pl.ds(s, 16)]
        q0 = _round_bf16(v0)
        q1 = _round_bf16(v1)
        q2 = _round_bf16(v2)
        yr[0, pl.ds(s, 16)] = q0 + q0
        yr[1, pl.ds(s, 16)] = q1 + q1
        yr[2, pl.ds(s, 16)] = q2 + q2
        y2b[pl.ds(s, 16)] = v0 * v0 + v1 * v1 + v2 * v2
        colp[pl.ds(s, 16)] = _INF16()
        return carry

    lax.fori_loop(0, _N // 16, prep_y, 0)

    def prep_x(k, carry):
        s = k * 16
        v0 = xv[0, pl.ds(r0 + s, 16)]
        v1 = xv[1, pl.ds(r0 + s, 16)]
        v2 = xv[2, pl.ds(r0 + s, 16)]
        xr[0, pl.ds(s, 16)] = _round_bf16(v0)
        xr[1, pl.ds(s, 16)] = _round_bf16(v1)
        xr[2, pl.ds(s, 16)] = _round_bf16(v2)
        x2b[pl.ds(s, 16)] = v0 * v0 + v1 * v1 + v2 * v2
        return carry

    lax.fori_loop(0, _RPT // 16, prep_x, 0)

    lanes = lax.iota(jnp.int32, 16)
    xors = [lanes ^ jnp.int32(k) for k in (8, 4, 2, 1)]

    rowsum = jnp.zeros((16,), jnp.float32)
    for half in range(_HALVES):
        xbase = half * 16 * _G
        xs = []
        x2s = []
        for gi in range(_G):
            off = xbase + gi * 16
            xs.append((xr[0, pl.ds(off, 16)],
                       xr[1, pl.ds(off, 16)],
                       xr[2, pl.ds(off, 16)]))
            x2s.append(x2b[pl.ds(off, 16)])

        def mbody(mc, accs, xs=xs, x2s=x2s):
            s = mc * 16
            o0 = yr[0, pl.ds(s, 16)]
            o1 = yr[1, pl.ds(s, 16)]
            o2 = yr[2, pl.ds(s, 16)]
            oy2 = y2b[pl.ds(s, 16)]
            accs = list(accs)
            colv = _INF16()
            for j in range(16):
                jc = jnp.full((16,), j, jnp.int32)
                b0 = _take16(o0, jc)
                b1 = _take16(o1, jc)
                b2 = _take16(o2, jc)
                y2v = _take16(oy2, jc)
                parts = []
                for gi in range(_G):
                    q0, q1, q2 = xs[gi]
                    t = q0 * b0 + q1 * b1 + q2 * b2
                    dd = (x2s[gi] + y2v) - t
                    accs[gi] = jnp.minimum(accs[gi], dd)
                    parts.append(dd)
                p = _tree_min(parts)
                for xk in xors:
                    p = jnp.minimum(p, _take16(p, xk))
                colv = jnp.where(lanes == jnp.int32(j), p, colv)
            colp[pl.ds(s, 16)] = jnp.minimum(colp[pl.ds(s, 16)], colv)
            return tuple(accs)

        accs = lax.fori_loop(0, _N // 16, mbody,
                             tuple(_INF16() for _ in range(_G)))
        for gi in range(_G):
            rowsum = rowsum + _sqrt16(accs[gi])

    pltpu.sync_copy(colp, shared.at[sid])
    plsc.subcore_barrier()
    sbase = (sid // 8) * 8
    m0 = slot * _RPT
    pltpu.sync_copy(shared.at[pl.ds(sbase, 8), pl.ds(m0, _RPT)], tmp8)

    colsum = jnp.zeros((16,), jnp.float32)
    for k in range(_RPT // 16):
        vs = [tmp8[p, pl.ds(k * 16, 16)] for p in range(8)]
        colsum = colsum + _sqrt16(_tree_min(vs))

    accv[0, :] = rowsum
    accv[1, :] = colsum
    pltpu.sync_copy(accv, out_hbm.at[cid * 16 + sid])


def _sc_chamfer(a1t, a2t):
    mesh = plsc.VectorSubcoreMesh(core_axis_name="c", subcore_axis_name="s")
    run = pl.kernel(
        _sc_body,
        out_type=jax.ShapeDtypeStruct((32, 2, 16), jnp.float32),
        mesh=mesh,
        scratch_types=[
            pltpu.VMEM((3, _N), jnp.float32),
            pltpu.VMEM((3, _N), jnp.float32),
            pltpu.VMEM((3, _N), jnp.float32),
            pltpu.VMEM((_N,), jnp.float32),
            pltpu.VMEM((3, _RPT), jnp.float32),
            pltpu.VMEM((_RPT,), jnp.float32),
            pltpu.VMEM((_N,), jnp.float32),
            pltpu.VMEM((8, _RPT), jnp.float32),
            pltpu.VMEM((2, 16), jnp.float32),
            pltpu.VMEM_SHARED((16, _N), jnp.float32),
        ],
    )
    return run(a1t, a2t)


def kernel(array1, array2):
    a1t = jnp.transpose(array1, (0, 2, 1))
    a2t = jnp.transpose(array2, (0, 2, 1))
    parts = _sc_chamfer(a1t, a2t)
    total = jnp.sum(parts)
    return total / jnp.float32(2 * _B * _N)
